# baseline (speedup 1.0000x reference)
import functools

import jax
import jax.numpy as jnp
from jax import lax
from jax.experimental import pallas as pl
from jax.experimental.pallas import tpu as pltpu
from jax.experimental.pallas import tpu_sc as plsc

_NUM_BINS = 50
_F = 6
_EMB = 64
_H1 = 256
_H2 = 128
_BT = 512

_NW = 16
_CH = 128


_SUP = 8
_CPS = 6


def _sc_gather(table_hbm, idx_hbm, out_hbm, idxb, rows, gsem, osem):
    wid = lax.axis_index("s") + lax.axis_index("c") * 16
    n_flat = idx_hbm.shape[0]
    rpw = n_flat // _NW
    sup_rows = _CH * _CPS
    base_w = wid * rpw
    iot = lax.iota(jnp.int32, 16)

    pltpu.sync_copy(idx_hbm.at[pl.ds(base_w, rpw)], idxb)

    def modloop(j, _):
        v = idxb[pl.ds(j * 16, 16)]
        pos = j * 16 + iot
        f = lax.rem(pos, _F)
        idxb[pl.ds(j * 16, 16)] = lax.rem(v, _NUM_BINS) + _NUM_BINS * f
        return 0


    out_copies = [None, None]
    for s in range(_SUP):
        h = s % 2
        if out_copies[h] is not None:
            out_copies[h].wait()
        gathers = []
        for j in range(_CPS):
            gathers.append(pltpu.async_copy(
                table_hbm.at[idxb.at[pl.ds(s * sup_rows + j * _CH, _CH)]],
                rows.at[h, pl.ds(j * _CH, _CH)], gsem))
        for g in gathers:
            g.wait()
        out_copies[h] = pltpu.async_copy(
            rows.at[h], out_hbm.at[pl.ds(base_w + s * sup_rows, sup_rows)],
            osem)
    for oc in out_copies:
        oc.wait()


def _dense_kernel(x_ref, W1_ref, b1_ref, W2_ref, b2_ref, out_ref):
    h1 = jnp.dot(x_ref[...], W1_ref[...], preferred_element_type=jnp.float32)
    h1 = h1 + b1_ref[...]
    h1 = jnp.where(h1 >= 0, h1, 0.2 * h1)
    h2 = jnp.dot(h1, W2_ref[...], preferred_element_type=jnp.float32)
    h2 = h2 + b2_ref[...]
    out_ref[...] = jnp.where(h2 >= 0, h2, 0.2 * h2)


def kernel(year, month, day, day_of_week, hour, minute,
           emb_year, emb_month, emb_day, emb_day_of_week, emb_hour,
           emb_minute, W1, b1, W2, b2):
    B = year.shape[0]
    raw = jnp.stack([year, month, day, day_of_week, hour, minute],
                    axis=1).astype(jnp.int32).reshape(-1)
    table = jnp.concatenate([emb_year, emb_month, emb_day, emb_day_of_week,
                             emb_hour, emb_minute], axis=0)

    mesh = plsc.VectorSubcoreMesh(core_axis_name="c", subcore_axis_name="s", num_cores=1)
    gathered = pl.kernel(
        _sc_gather,
        out_type=jax.ShapeDtypeStruct((B * _F, _EMB), jnp.float32),
        mesh=mesh,
        scratch_types=[
            pltpu.VMEM((B * _F // _NW,), jnp.int32),
            pltpu.VMEM((2, _CH * _CPS, _EMB), jnp.float32),
            pltpu.SemaphoreType.DMA,
            pltpu.SemaphoreType.DMA,
        ],
        compiler_params=pltpu.CompilerParams(use_tc_tiling_on_sc=False),
    )(table, raw)

    return gathered[: B, : _H2]
    x = gathered.reshape(B, _F * _EMB)
    grid = B // _BT
    out = pl.pallas_call(
        _dense_kernel,
        grid=(grid,),
        in_specs=[
            pl.BlockSpec((_BT, _F * _EMB), lambda i: (i, 0)),
            pl.BlockSpec(W1.shape, lambda i: (0, 0)),
            pl.BlockSpec((1, _H1), lambda i: (0, 0)),
            pl.BlockSpec(W2.shape, lambda i: (0, 0)),
            pl.BlockSpec((1, _H2), lambda i: (0, 0)),
        ],
        out_specs=pl.BlockSpec((_BT, _H2), lambda i: (i, 0)),
        out_shape=jax.ShapeDtypeStruct((B, _H2), jnp.float32),
    )(x, W1, b1.reshape(1, _H1), W2, b2.reshape(1, _H2))
    return out


# speedup vs baseline: 1.2933x; 1.2933x over previous
import functools

import jax
import jax.numpy as jnp
from jax import lax
from jax.experimental import pallas as pl
from jax.experimental.pallas import tpu as pltpu
from jax.experimental.pallas import tpu_sc as plsc

_NUM_BINS = 50
_F = 6
_EMB = 64
_H1 = 256
_H2 = 128
_BT = 512

_NW = 32
_CH = 768


_SUP = 4
_CPS = 1


def _sc_gather(table_hbm, idx_hbm, out_hbm, idxb, rows, gsem, osem):
    wid = lax.axis_index("s") * 2 + lax.axis_index("c")
    n_flat = idx_hbm.shape[0]
    rpw = n_flat // _NW
    sup_rows = _CH * _CPS
    base_w = wid * rpw
    iot = lax.iota(jnp.int32, 16)

    pltpu.sync_copy(idx_hbm.at[pl.ds(base_w, rpw)], idxb)

    def modloop(j, _):
        v = idxb[pl.ds(j * 16, 16)]
        pos = j * 16 + iot
        f = lax.rem(pos, _F)
        idxb[pl.ds(j * 16, 16)] = lax.rem(v, _NUM_BINS) + _NUM_BINS * f
        return 0

    lax.fori_loop(0, rpw // 16, modloop, 0)

    out_copies = [None, None]
    for s in range(_SUP):
        h = s % 2
        if out_copies[h] is not None:
            out_copies[h].wait()
        gathers = []
        for j in range(_CPS):
            gathers.append(pltpu.async_copy(
                table_hbm.at[idxb.at[pl.ds(s * sup_rows + j * _CH, _CH)]],
                rows.at[h, pl.ds(j * _CH, _CH)], gsem))
        for g in gathers:
            g.wait()
        out_copies[h] = pltpu.async_copy(
            rows.at[h], out_hbm.at[pl.ds(base_w + s * sup_rows, sup_rows)],
            osem)
    for oc in out_copies:
        oc.wait()


def _dense_kernel(x_ref, W1_ref, b1_ref, W2_ref, b2_ref, out_ref):
    h1 = jnp.dot(x_ref[...], W1_ref[...], preferred_element_type=jnp.float32)
    h1 = h1 + b1_ref[...]
    h1 = jnp.where(h1 >= 0, h1, 0.2 * h1)
    h2 = jnp.dot(h1, W2_ref[...], preferred_element_type=jnp.float32)
    h2 = h2 + b2_ref[...]
    out_ref[...] = jnp.where(h2 >= 0, h2, 0.2 * h2)


def kernel(year, month, day, day_of_week, hour, minute,
           emb_year, emb_month, emb_day, emb_day_of_week, emb_hour,
           emb_minute, W1, b1, W2, b2):
    B = year.shape[0]
    raw = jnp.stack([year, month, day, day_of_week, hour, minute],
                    axis=1).astype(jnp.int32).reshape(-1)
    table = jnp.concatenate([emb_year, emb_month, emb_day, emb_day_of_week,
                             emb_hour, emb_minute], axis=0)

    mesh = plsc.VectorSubcoreMesh(core_axis_name="c", subcore_axis_name="s")
    gathered = pl.kernel(
        _sc_gather,
        out_type=jax.ShapeDtypeStruct((B * _F, _EMB), jnp.float32),
        mesh=mesh,
        scratch_types=[
            pltpu.VMEM((B * _F // _NW,), jnp.int32),
            pltpu.VMEM((2, _CH * _CPS, _EMB), jnp.float32),
            pltpu.SemaphoreType.DMA,
            pltpu.SemaphoreType.DMA,
        ],
        compiler_params=pltpu.CompilerParams(use_tc_tiling_on_sc=False),
    )(table, raw)

    x = gathered.reshape(B, _F * _EMB)
    grid = B // _BT
    out = pl.pallas_call(
        _dense_kernel,
        grid=(grid,),
        in_specs=[
            pl.BlockSpec((_BT, _F * _EMB), lambda i: (i, 0)),
            pl.BlockSpec(W1.shape, lambda i: (0, 0)),
            pl.BlockSpec((1, _H1), lambda i: (0, 0)),
            pl.BlockSpec(W2.shape, lambda i: (0, 0)),
            pl.BlockSpec((1, _H2), lambda i: (0, 0)),
        ],
        out_specs=pl.BlockSpec((_BT, _H2), lambda i: (i, 0)),
        out_shape=jax.ShapeDtypeStruct((B, _H2), jnp.float32),
    )(x, W1, b1.reshape(1, _H1), W2, b2.reshape(1, _H2))
    return out
